# baseline (speedup 1.0000x reference)
import functools

import jax
import jax.numpy as jnp
from jax import lax
from jax.experimental import pallas as pl
from jax.experimental.pallas import tpu as pltpu
from jax.experimental.pallas import tpu_sc as plsc

L = 16
NC, NS = 2, 16
NW = NC * NS
TABLE_SIZE = 128
N_CHUNKS = 16
IN_DEPTH = 4
OUT_DEPTH = 4


def _make_sc_lookup(n_total: int, n_keys: int):
    per_w = n_total // NW
    chunk = per_w // N_CHUNKS
    mesh = plsc.VectorSubcoreMesh(core_axis_name="c", subcore_axis_name="s")

    @functools.partial(
        pl.kernel,
        mesh=mesh,
        out_type=jax.ShapeDtypeStruct((n_total,), jnp.int32),
        scratch_types=[
            [pltpu.VMEM((chunk,), jnp.int32) for _ in range(IN_DEPTH)],
            [pltpu.VMEM((chunk,), jnp.int32) for _ in range(OUT_DEPTH)],
            pltpu.VMEM((TABLE_SIZE,), jnp.int32),
            pltpu.VMEM((TABLE_SIZE,), jnp.int32),
            pltpu.VMEM((TABLE_SIZE,), jnp.int32),
            [pltpu.SemaphoreType.DMA for _ in range(IN_DEPTH)],
            [pltpu.SemaphoreType.DMA for _ in range(OUT_DEPTH)],
        ],
        compiler_params=pltpu.CompilerParams(needs_layout_passes=False),
    )
    def lookup(ids_hbm, keys_hbm, vals_hbm, out_hbm,
               ibufs, obufs, inv, kbuf, vbuf, in_sems, out_sems):
        wid = lax.axis_index("s") * NC + lax.axis_index("c")
        base = wid * per_w

        def in_copy_d(c, slot):
            return pltpu.make_async_copy(
                ids_hbm.at[pl.ds(base + c * chunk, chunk)],
                ibufs[slot], in_sems[slot])

        def out_copy_d(c, slot):
            return pltpu.make_async_copy(
                obufs[slot],
                out_hbm.at[pl.ds(base + c * chunk, chunk)],
                out_sems[slot])

        for c in range(IN_DEPTH):
            in_copy_d(c, c).start()
        for i in range(TABLE_SIZE // L):
            sl = pl.ds(i * L, L)
            kbuf[sl] = jnp.full((L,), TABLE_SIZE - 1, jnp.int32)
            vbuf[sl] = jnp.zeros((L,), jnp.int32)
            inv[sl] = jnp.zeros((L,), jnp.int32)
        pltpu.sync_copy(keys_hbm, kbuf.at[pl.ds(0, n_keys)])
        pltpu.sync_copy(vals_hbm, vbuf.at[pl.ds(0, n_keys)])
        for i in range(TABLE_SIZE // L):
            sl = pl.ds(i * L, L)
            plsc.store_scatter(inv, [kbuf[sl]], vbuf[sl])

        @pl.loop(0, N_CHUNKS, step=IN_DEPTH)
        def _(g):
            for j in range(IN_DEPTH):
                c = g + j
                in_copy_d(c, j).wait()
                @pl.when(c >= OUT_DEPTH)
                def _():
                    out_copy_d(c - OUT_DEPTH, j).wait()
                ibuf, obuf = ibufs[j], obufs[j]

                @plsc.parallel_loop(0, chunk, step=L, unroll=8)
                def _(off):
                    sl = pl.ds(off, L)
                    obuf[sl] = plsc.load_gather(inv, [ibuf[sl]])

                out_copy_d(c, j).start()
                @pl.when(c + IN_DEPTH < N_CHUNKS)
                def _():
                    in_copy_d(c + IN_DEPTH, j).start()

        for j in range(OUT_DEPTH):
            out_copy_d(N_CHUNKS - OUT_DEPTH + j, j).wait()

    return lookup


def kernel(inputs, keys, vals):
    batch, hist = inputs.shape
    n_total = batch * hist
    ht, bt = hist // 8, batch // 128
    flat = (inputs.T.reshape(ht, 8, bt, 128)
            .transpose(0, 2, 1, 3).reshape(n_total))
    out = _make_sc_lookup(n_total, keys.shape[0])(
        flat, keys.astype(jnp.int32), vals.astype(jnp.int32))
    return (out.reshape(ht, bt, 8, 128).transpose(0, 2, 1, 3)
            .reshape(hist, batch).T)
